# baseline (speedup 1.0000x reference)
import functools

import jax
import jax.numpy as jnp
from jax import lax
from jax.experimental import pallas as pl
from jax.experimental.pallas import tpu as pltpu
from jax.experimental.pallas import tpu_sc as plsc

_DIM = 64
_VOCAB = 100000
_BATCH = 16384
_NNEG = 20
_NPAIR = _NNEG + 1
_NW = 32
_CH = 128
_NB = 6
_NSLICE = 2
_BS = _BATCH // _NSLICE

_LOG2E = 1.4426950408889634

_POLY = (-0.3665129829491377, 0.7213459840780112, -0.07983222595229246,
         -0.004957223416339075, 0.0023628927052507724, 0.00022657838744351012,
         -0.00011899139943125192)


def _f_poly(t):
    acc = jnp.full_like(t, _POLY[-1])
    for c in _POLY[-2::-1]:
        acc = acc * t + c
    return acc


def _sigmoid(x):
    return 1.0 / (1.0 + jnp.exp2(x * -_LOG2E))


def _pass0_body(w_ref, zZ_ref):
    s = _sigmoid(w_ref[...])
    r = pltpu.roll(s, _DIM, axis=1)
    lane = lax.broadcasted_iota(jnp.int32, s.shape, 1)
    Zf = r + s * (1.0 - r)
    zZ_ref[...] = jnp.where(lane < _DIM, s, Zf)


def _pass0(W_ctx, b0=800):
    return pl.pallas_call(
        _pass0_body,
        grid=(_VOCAB // b0,),
        in_specs=[pl.BlockSpec((b0, 2 * _DIM), lambda i: (i, 0))],
        out_specs=pl.BlockSpec((b0, 2 * _DIM), lambda i: (i, 0)),
        out_shape=jax.ShapeDtypeStruct((_VOCAB, 2 * _DIM), jnp.float32),
    )(W_ctx)


def _sc_gather_body(w_word, zZ_ctx, idx_u, idx_c, out_u, out_c, *scratch):
    idx_bufs = scratch[0:_NB]
    row_bufs = scratch[_NB:2 * _NB]
    isems = scratch[2 * _NB:3 * _NB]
    osems = scratch[3 * _NB:4 * _NB]
    wid = lax.axis_index("s") * 2 + lax.axis_index("c")
    u_per_w = _BS // _NW
    c_per_w = _NPAIR * _BS // _NW

    def u_group(g, carry):
        gathers = []
        for b in range(2):
            off = wid * u_per_w + b * _CH
            pltpu.sync_copy(idx_u.at[pl.ds(off, _CH)], idx_bufs[b])
            gathers.append(pltpu.async_copy(w_word.at[idx_bufs[b]],
                                            row_bufs[b], isems[b]))
        outs = []
        for b in range(2):
            gathers[b].wait()
            off = wid * u_per_w + b * _CH
            outs.append(pltpu.async_copy(row_bufs[b],
                                         out_u.at[pl.ds(off, _CH)], osems[b]))
        for b in range(2):
            outs[b].wait()
        return carry

    lax.fori_loop(0, 1, u_group, 0)

    def c_group(g, carry):
        gathers = []
        for b in range(_NB):
            off = wid * c_per_w + (g * _NB + b) * _CH
            pltpu.sync_copy(idx_c.at[pl.ds(off, _CH)], idx_bufs[b])
            gathers.append(pltpu.async_copy(zZ_ctx.at[idx_bufs[b]],
                                            row_bufs[b], isems[b]))
        outs = []
        for b in range(_NB):
            off = wid * c_per_w + (g * _NB + b) * _CH
            gathers[b].wait()
            outs.append(pltpu.async_copy(row_bufs[b],
                                         out_c.at[pl.ds(off, _CH)], osems[b]))
        for o in outs:
            o.wait()
        return carry

    lax.fori_loop(0, c_per_w // (_CH * _NB), c_group, 0)


@functools.cache
def _sc_gather():
    return pl.kernel(
        _sc_gather_body,
        out_type=(
            jax.ShapeDtypeStruct((_BS, 2 * _DIM), jnp.float32),
            jax.ShapeDtypeStruct((_NPAIR * _BS, 2 * _DIM), jnp.float32),
        ),
        mesh=plsc.VectorSubcoreMesh(core_axis_name="c", subcore_axis_name="s"),
        scratch_types=(
            [pltpu.VMEM((_CH,), jnp.int32) for _ in range(_NB)]
            + [pltpu.VMEM((_CH, 2 * _DIM), jnp.float32) for _ in range(_NB)]
            + [pltpu.SemaphoreType.DMA for _ in range(2 * _NB)]
        ),
    )


def _tc_body(u_ref, c_ref, vols_ref, ints_ref, tv_ref, zZu_ref):
    j = pl.program_id(1)

    @pl.when(j == 0)
    def _():
        s = _sigmoid(u_ref[...].T)
        zu0 = s[:_DIM]
        Zu0 = zu0 + s[_DIM:] * (1.0 - zu0)
        zZu_ref[:_DIM] = zu0
        zZu_ref[_DIM:] = Zu0
        tv_ref[0, 0, :] = jnp.sum(_f_poly(Zu0 - zu0), axis=0)

    zu = zZu_ref[:_DIM]
    Zu = zZu_ref[_DIM:]
    cT = c_ref[...].T
    zc = cT[:_DIM]
    Zc = cT[_DIM:]
    t = jnp.concatenate(
        [Zc - zc, jnp.minimum(Zc, Zu) - jnp.maximum(zc, zu)], axis=0)
    f = _f_poly(t)
    vols_ref[0, 0, :] = jnp.sum(f[:_DIM], axis=0)
    ints_ref[0, 0, :] = jnp.sum(f[_DIM:], axis=0)


def _tc_compute(u_rows, ctx_rows, bb=1024):
    nb = _BS // bb
    return pl.pallas_call(
        _tc_body,
        grid=(nb, _NPAIR),
        in_specs=[
            pl.BlockSpec((bb, 2 * _DIM), lambda i, j: (i, 0)),
            pl.BlockSpec((bb, 2 * _DIM), lambda i, j, nb=nb: (j * nb + i, 0)),
        ],
        out_specs=[
            pl.BlockSpec((1, 1, bb), lambda i, j: (j, 0, i)),
            pl.BlockSpec((1, 1, bb), lambda i, j: (j, 0, i)),
            pl.BlockSpec((1, 1, bb), lambda i, j: (0, 0, i)),
        ],
        out_shape=[
            jax.ShapeDtypeStruct((_NPAIR, 1, _BS), jnp.float32),
            jax.ShapeDtypeStruct((_NPAIR, 1, _BS), jnp.float32),
            jax.ShapeDtypeStruct((1, 1, _BS), jnp.float32),
        ],
        scratch_shapes=[pltpu.VMEM((2 * _DIM, bb), jnp.float32)],
    )(u_rows, ctx_rows)


def kernel(pos_u, pos_w, neg_w, W_word, W_ctx):
    pos_u = pos_u.astype(jnp.int32)
    pos_w = pos_w.astype(jnp.int32)
    neg_w = neg_w.astype(jnp.int32)
    zZ_ctx = _pass0(W_ctx)
    vols_l, ints_l, tv_l = [], [], []
    for k in range(_NSLICE):
        sl = slice(k * _BS, (k + 1) * _BS)
        idx_ctx = jnp.concatenate([neg_w[sl].T.reshape(-1), pos_w[sl]])
        u_rows, ctx_rows = _sc_gather()(W_word, zZ_ctx, pos_u[sl], idx_ctx)
        vols, ints, tv = _tc_compute(u_rows, ctx_rows)
        vols_l.append(vols[:, 0, :])
        ints_l.append(ints[:, 0, :])
        tv_l.append(tv[0, 0, :])
    vols = jnp.concatenate(vols_l, axis=1)
    ints = jnp.concatenate(ints_l, axis=1)
    tv = jnp.concatenate(tv_l)
    return (tv, vols[_NNEG], vols[:_NNEG].T, ints[_NNEG], ints[:_NNEG].T)


# speedup vs baseline: 1.2122x; 1.2122x over previous
import functools

import jax
import jax.numpy as jnp
from jax import lax
from jax.experimental import pallas as pl
from jax.experimental.pallas import tpu as pltpu
from jax.experimental.pallas import tpu_sc as plsc

_DIM = 64
_BATCH = 16384
_NNEG = 20
_NPAIR = _NNEG + 1
_NW = 32
_CH = 128
_NSLICE = 4
_BS = _BATCH // _NSLICE

_LOG2E = 1.4426950408889634

_POLY = (-0.3665129829491377, 0.7213459840780112, -0.07983222595229246,
         -0.004957223416339075, 0.0023628927052507724, 0.00022657838744351012,
         -0.00011899139943125192)


def _f_poly(t):
    acc = jnp.full_like(t, _POLY[-1])
    for c in _POLY[-2::-1]:
        acc = acc * t + c
    return acc


def _sigmoid(x):
    return 1.0 / (1.0 + jnp.exp2(x * -_LOG2E))


def _gather_loop(table, idx_hbm, out_hbm, base, ngroups, nb, idx_bufs,
                 row_bufs, isems, osems):
    def group(g, carry):
        gathers = []
        for b in range(nb):
            off = base + (g * nb + b) * _CH
            pltpu.sync_copy(idx_hbm.at[pl.ds(off, _CH)], idx_bufs[b])
            gathers.append(pltpu.async_copy(table.at[idx_bufs[b]], row_bufs[b],
                                            isems[b]))
        outs = []
        for b in range(nb):
            gathers[b].wait()
            off = base + (g * nb + b) * _CH
            outs.append(pltpu.async_copy(row_bufs[b],
                                         out_hbm.at[pl.ds(off, _CH)], osems[b]))
        for b in range(nb):
            outs[b].wait()
        return carry

    lax.fori_loop(0, ngroups, group, 0)


_NB = 7


def _sc_gather_body(w_word, w_ctx, idx_u, idx_c, out_u, out_c, *scratch):
    idx_bufs = scratch[0:_NB]
    row_bufs = scratch[_NB:2 * _NB]
    isems = scratch[2 * _NB:3 * _NB]
    osems = scratch[3 * _NB:4 * _NB]
    wid = lax.axis_index("s") * 2 + lax.axis_index("c")
    u_per_w = _BS // _NW
    c_per_w = _NPAIR * _BS // _NW
    _gather_loop(w_word, idx_u, out_u, wid * u_per_w, 1, u_per_w // _CH,
                 idx_bufs, row_bufs, isems, osems)
    _gather_loop(w_ctx, idx_c, out_c, wid * c_per_w, c_per_w // (_CH * _NB),
                 _NB, idx_bufs, row_bufs, isems, osems)


@functools.cache
def _sc_gather():
    return pl.kernel(
        _sc_gather_body,
        out_type=(
            jax.ShapeDtypeStruct((_BS, 2 * _DIM), jnp.float32),
            jax.ShapeDtypeStruct((_NPAIR * _BS, 2 * _DIM), jnp.float32),
        ),
        mesh=plsc.VectorSubcoreMesh(core_axis_name="c", subcore_axis_name="s"),
        scratch_types=(
            [pltpu.VMEM((_CH,), jnp.int32) for _ in range(_NB)]
            + [pltpu.VMEM((_CH, 2 * _DIM), jnp.float32) for _ in range(_NB)]
            + [pltpu.SemaphoreType.DMA for _ in range(2 * _NB)]
        ),
    )


def _box_t(x):
    s = _sigmoid(x.T)
    z = s[:_DIM]
    Z = z + s[_DIM:] * (1.0 - z)
    return z, Z


def _tc_body(u_ref, c_ref, vols_ref, ints_ref, tv_ref, zZu_ref):
    j = pl.program_id(1)

    @pl.when(j == 0)
    def _():
        zu0, Zu0 = _box_t(u_ref[...])
        zZu_ref[:_DIM] = zu0
        zZu_ref[_DIM:] = Zu0
        tv_ref[0, 0, :] = jnp.sum(_f_poly(Zu0 - zu0), axis=0)

    zu = zZu_ref[:_DIM]
    Zu = zZu_ref[_DIM:]
    zc, Zc = _box_t(c_ref[...])
    t = jnp.concatenate(
        [Zc - zc, jnp.minimum(Zc, Zu) - jnp.maximum(zc, zu)], axis=0)
    f = _f_poly(t)
    vols_ref[0, 0, :] = jnp.sum(f[:_DIM], axis=0)
    ints_ref[0, 0, :] = jnp.sum(f[_DIM:], axis=0)


def _tc_compute(u_rows, ctx_rows, bb=1024):
    nb = _BS // bb
    return pl.pallas_call(
        _tc_body,
        grid=(nb, _NPAIR),
        in_specs=[
            pl.BlockSpec((bb, 2 * _DIM), lambda i, j: (i, 0)),
            pl.BlockSpec((bb, 2 * _DIM), lambda i, j, nb=nb: (j * nb + i, 0)),
        ],
        out_specs=[
            pl.BlockSpec((1, 1, bb), lambda i, j: (j, 0, i)),
            pl.BlockSpec((1, 1, bb), lambda i, j: (j, 0, i)),
            pl.BlockSpec((1, 1, bb), lambda i, j: (0, 0, i)),
        ],
        out_shape=[
            jax.ShapeDtypeStruct((_NPAIR, 1, _BS), jnp.float32),
            jax.ShapeDtypeStruct((_NPAIR, 1, _BS), jnp.float32),
            jax.ShapeDtypeStruct((1, 1, _BS), jnp.float32),
        ],
        scratch_shapes=[pltpu.VMEM((2 * _DIM, bb), jnp.float32)],
    )(u_rows, ctx_rows)


def kernel(pos_u, pos_w, neg_w, W_word, W_ctx):
    pos_u = pos_u.astype(jnp.int32)
    pos_w = pos_w.astype(jnp.int32)
    neg_w = neg_w.astype(jnp.int32)
    vols_l, ints_l, tv_l = [], [], []
    for k in range(_NSLICE):
        sl = slice(k * _BS, (k + 1) * _BS)
        idx_ctx = jnp.concatenate([neg_w[sl].T.reshape(-1), pos_w[sl]])
        u_rows, ctx_rows = _sc_gather()(W_word, W_ctx, pos_u[sl], idx_ctx)
        vols, ints, tv = _tc_compute(u_rows, ctx_rows)
        vols_l.append(vols[:, 0, :])
        ints_l.append(ints[:, 0, :])
        tv_l.append(tv[0, 0, :])
    vols = jnp.concatenate(vols_l, axis=1)
    ints = jnp.concatenate(ints_l, axis=1)
    tv = jnp.concatenate(tv_l)
    return (tv, vols[_NNEG], vols[:_NNEG].T, ints[_NNEG], ints[:_NNEG].T)


# speedup vs baseline: 1.2694x; 1.0473x over previous
import functools

import jax
import jax.numpy as jnp
from jax import lax
from jax.experimental import pallas as pl
from jax.experimental.pallas import tpu as pltpu
from jax.experimental.pallas import tpu_sc as plsc

_DIM = 64
_BATCH = 16384
_NNEG = 20
_NPAIR = _NNEG + 1
_NW = 32
_CH = 128
_NSLICE = 4
_BS = _BATCH // _NSLICE

_LOG2E = 1.4426950408889634

_POLY = (-0.3665167014303693, 0.7213459840780102, -0.07976529329011446,
         -0.004957223416335807, 0.002184405606105031, 0.00022657838744066794)


def _f_poly(t):
    acc = jnp.full_like(t, _POLY[-1])
    for c in _POLY[-2::-1]:
        acc = acc * t + c
    return acc


def _sigmoid(x):
    return 1.0 / (1.0 + jnp.exp2(x * -_LOG2E))


def _gather_loop(table, idx_hbm, out_hbm, base, ngroups, nb, idx_bufs,
                 row_bufs, isems, osems):
    def group(g, carry):
        gathers = []
        for b in range(nb):
            off = base + (g * nb + b) * _CH
            pltpu.sync_copy(idx_hbm.at[pl.ds(off, _CH)], idx_bufs[b])
            gathers.append(pltpu.async_copy(table.at[idx_bufs[b]], row_bufs[b],
                                            isems[b]))
        outs = []
        for b in range(nb):
            gathers[b].wait()
            off = base + (g * nb + b) * _CH
            outs.append(pltpu.async_copy(row_bufs[b],
                                         out_hbm.at[pl.ds(off, _CH)], osems[b]))
        for b in range(nb):
            outs[b].wait()
        return carry

    lax.fori_loop(0, ngroups, group, 0)


_NB = 7


def _sc_gather_body(w_word, w_ctx, idx_u, idx_c, out_u, out_c, *scratch):
    idx_bufs = scratch[0:_NB]
    row_bufs = scratch[_NB:2 * _NB]
    isems = scratch[2 * _NB:3 * _NB]
    osems = scratch[3 * _NB:4 * _NB]
    wid = lax.axis_index("s") * 2 + lax.axis_index("c")
    u_per_w = _BS // _NW
    c_per_w = _NPAIR * _BS // _NW
    _gather_loop(w_word, idx_u, out_u, wid * u_per_w, 1, u_per_w // _CH,
                 idx_bufs, row_bufs, isems, osems)
    _gather_loop(w_ctx, idx_c, out_c, wid * c_per_w, c_per_w // (_CH * _NB),
                 _NB, idx_bufs, row_bufs, isems, osems)


@functools.cache
def _sc_gather():
    return pl.kernel(
        _sc_gather_body,
        out_type=(
            jax.ShapeDtypeStruct((_BS, 2 * _DIM), jnp.float32),
            jax.ShapeDtypeStruct((_NPAIR * _BS, 2 * _DIM), jnp.float32),
        ),
        mesh=plsc.VectorSubcoreMesh(core_axis_name="c", subcore_axis_name="s"),
        scratch_types=(
            [pltpu.VMEM((_CH,), jnp.int32) for _ in range(_NB)]
            + [pltpu.VMEM((_CH, 2 * _DIM), jnp.float32) for _ in range(_NB)]
            + [pltpu.SemaphoreType.DMA for _ in range(2 * _NB)]
        ),
    )


def _box_t(x):
    s = _sigmoid(x).T
    z = s[:_DIM]
    Z = z + s[_DIM:] * (1.0 - z)
    return z, Z


def _tc_body(u_ref, c_ref, vols_ref, ints_ref, tv_ref, zZu_ref):
    j = pl.program_id(1)

    @pl.when(j == 0)
    def _():
        zu0, Zu0 = _box_t(u_ref[...])
        zZu_ref[:_DIM] = zu0
        zZu_ref[_DIM:] = Zu0
        tv_ref[0, 0, :] = jnp.sum(_f_poly(Zu0 - zu0), axis=0)

    zu = zZu_ref[:_DIM]
    Zu = zZu_ref[_DIM:]
    zc, Zc = _box_t(c_ref[...])
    fv = _f_poly(Zc - zc)
    fi = _f_poly(jnp.minimum(Zc, Zu) - jnp.maximum(zc, zu))
    vols_ref[0, 0, :] = jnp.sum(fv, axis=0)
    ints_ref[0, 0, :] = jnp.sum(fi, axis=0)


def _tc_compute(u_rows, ctx_rows, bb=1024):
    nb = _BS // bb
    return pl.pallas_call(
        _tc_body,
        grid=(nb, _NPAIR),
        in_specs=[
            pl.BlockSpec((bb, 2 * _DIM), lambda i, j: (i, 0)),
            pl.BlockSpec((bb, 2 * _DIM), lambda i, j, nb=nb: (j * nb + i, 0)),
        ],
        out_specs=[
            pl.BlockSpec((1, 1, bb), lambda i, j: (j, 0, i)),
            pl.BlockSpec((1, 1, bb), lambda i, j: (j, 0, i)),
            pl.BlockSpec((1, 1, bb), lambda i, j: (0, 0, i)),
        ],
        out_shape=[
            jax.ShapeDtypeStruct((_NPAIR, 1, _BS), jnp.float32),
            jax.ShapeDtypeStruct((_NPAIR, 1, _BS), jnp.float32),
            jax.ShapeDtypeStruct((1, 1, _BS), jnp.float32),
        ],
        scratch_shapes=[pltpu.VMEM((2 * _DIM, bb), jnp.float32)],
    )(u_rows, ctx_rows)


def kernel(pos_u, pos_w, neg_w, W_word, W_ctx):
    pos_u = pos_u.astype(jnp.int32)
    pos_w = pos_w.astype(jnp.int32)
    neg_w = neg_w.astype(jnp.int32)
    vols_l, ints_l, tv_l = [], [], []
    for k in range(_NSLICE):
        sl = slice(k * _BS, (k + 1) * _BS)
        idx_ctx = jnp.concatenate([neg_w[sl].T.reshape(-1), pos_w[sl]])
        u_rows, ctx_rows = _sc_gather()(W_word, W_ctx, pos_u[sl], idx_ctx)
        vols, ints, tv = _tc_compute(u_rows, ctx_rows)
        vols_l.append(vols[:, 0, :])
        ints_l.append(ints[:, 0, :])
        tv_l.append(tv[0, 0, :])
    vols = jnp.concatenate(vols_l, axis=1)
    ints = jnp.concatenate(ints_l, axis=1)
    tv = jnp.concatenate(tv_l)
    return (tv, vols[_NNEG], vols[:_NNEG].T, ints[_NNEG], ints[:_NNEG].T)
